# baseline (speedup 1.0000x reference)
import jax
import jax.numpy as jnp
import numpy as np
from jax.experimental import pallas as pl
from jax.experimental.pallas import tpu as pltpu

_SIGMA = 4.0
_R = int(np.ceil(3.0 * _SIGMA))
_N = 512
_BLK = 128


def _blur_matrix():
    x = np.arange(-_R, _R + 1, dtype=np.float64)
    w = np.exp(-0.5 * (x / _SIGMA) ** 2)
    w = w / w.sum()
    b = np.zeros((_N, _N), dtype=np.float64)
    rows = np.arange(_N)
    for t in range(2 * _R + 1):
        cols = np.clip(rows + t - _R, 0, _N - 1)
        np.add.at(b, (rows, cols), w[t])
    return b


_B = _blur_matrix()


def _blur_body(img_ref, b_ref, out_ref, s_ref):
    step = pl.program_id(0)
    b = b_ref[...]

    @pl.when(step < 4)
    def _phase_a():
        blk = img_ref[0].astype(jnp.bfloat16)
        s = jax.lax.dot_general(
            blk, b, (((1,), (1,)), ((), ())),
            preferred_element_type=jnp.float32)
        s_ref[pl.ds(step * _BLK, _BLK), :] = s

    @pl.when(step >= 4)
    def _phase_b():
        j = step - 4
        s_col = s_ref[:, pl.ds(j * _BLK, _BLK)].astype(jnp.bfloat16)
        out_ref[0] = jax.lax.dot(b, s_col, preferred_element_type=jnp.float32)


@jax.jit
def kernel(img):
    return pl.pallas_call(
        _blur_body,
        grid=(8,),
        in_specs=[
            pl.BlockSpec((1, _BLK, _N), lambda s: (0, jnp.minimum(s, 3), 0)),
            pl.BlockSpec((_N, _N), lambda s: (0, 0)),
        ],
        out_specs=pl.BlockSpec((1, _N, _BLK),
                               lambda s: (0, 0, jnp.maximum(s - 4, 0))),
        scratch_shapes=[pltpu.VMEM((_N, _N), jnp.float32)],
        out_shape=jax.ShapeDtypeStruct((1, _N, _N), jnp.float32),
    )(img, jnp.asarray(_B, dtype=jnp.bfloat16))


# speedup vs baseline: 1.8857x; 1.8857x over previous
import jax
import jax.numpy as jnp
import numpy as np
from jax.experimental import pallas as pl

_SIGMA = 4.0
_R = int(np.ceil(3.0 * _SIGMA))
_N = 512


def _blur_matrix():
    x = np.arange(-_R, _R + 1, dtype=np.float64)
    w = np.exp(-0.5 * (x / _SIGMA) ** 2)
    w = w / w.sum()
    b = np.zeros((_N, _N), dtype=np.float64)
    rows = np.arange(_N)
    for t in range(2 * _R + 1):
        cols = np.clip(rows + t - _R, 0, _N - 1)
        np.add.at(b, (rows, cols), w[t])
    return b


_B = _blur_matrix()


def _blur_body(img_ref, b_ref, out_ref):
    img = img_ref[0].astype(jnp.bfloat16)
    b = b_ref[...]
    tmp = jax.lax.dot(b, img, preferred_element_type=jnp.float32)
    out = jax.lax.dot_general(
        tmp.astype(jnp.bfloat16), b, (((1,), (1,)), ((), ())),
        preferred_element_type=jnp.float32)
    out_ref[0] = out


@jax.jit
def kernel(img):
    return pl.pallas_call(
        _blur_body,
        out_shape=jax.ShapeDtypeStruct((1, _N, _N), jnp.float32),
    )(img, jnp.asarray(_B, dtype=jnp.bfloat16))
